# baseline (speedup 1.0000x reference)
import jax
import jax.numpy as jnp
from jax import lax
from jax.experimental import pallas as pl
from jax.experimental.pallas import tpu as pltpu
from jax.experimental.pallas import tpu_sc as plsc

B = 16384
D = 64
NC = 2
NS = 16
NW = NC * NS
BPW = B // NW
PP = 17
G = 16
NG = BPW // G


def _bias_body(midx_hbm, uidx_hbm, mu_hbm, bm_hbm, bu_hbm, out_hbm,
               midx_v, uidx_v, bm_v, bu_v, mu_v, out_v, sem_bm, sem_bu):
    wid = lax.axis_index("s") * NC + lax.axis_index("c")
    base = wid * BPW
    pltpu.sync_copy(midx_hbm.at[pl.ds(base, BPW)], midx_v)
    pltpu.sync_copy(uidx_hbm.at[pl.ds(base, BPW)], uidx_v)
    pltpu.sync_copy(mu_hbm, mu_v.at[pl.ds(0, 1)])
    cbm = pltpu.async_copy(bm_hbm.at[midx_v], bm_v, sem_bm)
    cbu = pltpu.async_copy(bu_hbm.at[uidx_v], bu_v, sem_bu)
    cbm.wait()
    cbu.wait()
    mu_s = mu_v[...][0]

    def blk(j, _):
        rb = j * 16
        out_v[pl.ds(rb, 16)] = mu_s + (bm_v[pl.ds(rb, 16)] +
                                       bu_v[pl.ds(rb, 16)])
        return 0

    lax.fori_loop(0, BPW // 16, blk, 0, unroll=4)
    pltpu.sync_copy(out_v, out_hbm.at[pl.ds(base, BPW)])


def _dot_body(midx_hbm, uidx_hbm, wm_hbm, wu_hbm, pb_hbm, out_hbm,
              midx_v, uidx_v, tm_v, tu_v, pb_v, part_v, out_v,
              sem_m0, sem_m1, sem_u0, sem_u1):
    wid = lax.axis_index("s") * NC + lax.axis_index("c")
    base = wid * BPW

    pltpu.sync_copy(midx_hbm.at[pl.ds(base, BPW)], midx_v)
    pltpu.sync_copy(uidx_hbm.at[pl.ds(base, BPW)], uidx_v)
    pltpu.sync_copy(pb_hbm.at[pl.ds(base, BPW)], pb_v)

    sems_m = (sem_m0, sem_m1)
    sems_u = (sem_u0, sem_u1)

    def issue(g, slot):
        mv = midx_v[pl.ds(g * G, G)]
        uv = uidx_v[pl.ds(g * G, G)]
        for k in range(G):
            tm0 = lax.shift_right_logical(mv[k], 3) * 8
            tu0 = lax.shift_right_logical(uv[k], 3) * 8
            d0 = (slot * G + k) * 8
            pltpu.async_copy(wm_hbm.at[pl.ds(tm0, 8)],
                             tm_v.at[pl.ds(d0, 8)], sems_m[slot])
            pltpu.async_copy(wu_hbm.at[pl.ds(tu0, 8)],
                             tu_v.at[pl.ds(d0, 8)], sems_u[slot])

    def drain(slot):
        d0 = slot * G * 8
        pltpu.make_async_copy(wm_hbm.at[pl.ds(0, G * 8)],
                              tm_v.at[pl.ds(d0, G * 8)], sems_m[slot]).wait()
        pltpu.make_async_copy(wu_hbm.at[pl.ds(0, G * 8)],
                              tu_v.at[pl.ds(d0, G * 8)], sems_u[slot]).wait()

    def compute(g, slot):
        mv = midx_v[pl.ds(g * G, G)]
        uv = uidx_v[pl.ds(g * G, G)]
        for k in range(G):
            sm = (mv[k] & 7) + (slot * G + k) * 8
            su = (uv[k] & 7) + (slot * G + k) * 8
            a0 = tm_v[sm, pl.ds(0, 16)] * tu_v[su, pl.ds(0, 16)]
            a1 = tm_v[sm, pl.ds(16, 16)] * tu_v[su, pl.ds(16, 16)]
            a2 = tm_v[sm, pl.ds(32, 16)] * tu_v[su, pl.ds(32, 16)]
            a3 = tm_v[sm, pl.ds(48, 16)] * tu_v[su, pl.ds(48, 16)]
            part_v[g * 4 + k // 4, pl.ds((k % 4) * 16, 16)] = \
                (a0 + a1) + (a2 + a3)

    issue(0, 0)
    issue(1, 1)

    def grp(gg, _):
        g0 = gg * 2
        drain(0)
        compute(g0, 0)
        issue(g0 + 2, 0)
        drain(1)
        compute(g0 + 1, 1)
        issue(g0 + 3, 1)
        return 0

    lax.fori_loop(0, NG // 2 - 1, grp, 0, unroll=1)

    drain(0)
    compute(NG - 2, 0)
    drain(1)
    compute(NG - 1, 1)

    iota16 = lax.iota(jnp.int32, 16)
    zero16 = jnp.zeros((16,), jnp.int32)

    def blk(j, _):
        rb = j * 16
        rv = rb + iota16
        prow = lax.shift_right_logical(rv, 2)
        pcol = (rv & 3) * 16
        acc0 = pb_v[pl.ds(rb, 16)]
        acc2 = plsc.load_gather(part_v, [prow, pcol])
        acc3 = plsc.load_gather(part_v, [prow, pcol + 1])
        for c in range(2, 16, 2):
            acc2 = acc2 + plsc.load_gather(part_v, [prow, pcol + c])
            acc3 = acc3 + plsc.load_gather(part_v, [prow, pcol + (c + 1)])
        out_v[pl.ds(rb, 16)] = acc0 + (acc2 + acc3)
        return 0

    lax.fori_loop(0, BPW // 16, blk, 0, unroll=2)

    pltpu.sync_copy(out_v, out_hbm.at[pl.ds(base, BPW)])


@jax.jit
def kernel(x, W_movie, W_user, mu, b_movie, b_user):
    mesh = plsc.VectorSubcoreMesh(core_axis_name="c", subcore_axis_name="s",
                                  num_cores=NC, num_subcores=NS)
    bias = pl.kernel(
        _bias_body,
        out_type=jax.ShapeDtypeStruct((B,), jnp.float32),
        mesh=mesh,
        compiler_params=pltpu.CompilerParams(needs_layout_passes=False,
                                             use_tc_tiling_on_sc=False),
        scratch_types=[
            pltpu.VMEM((BPW,), jnp.int32),
            pltpu.VMEM((BPW,), jnp.int32),
            pltpu.VMEM((BPW,), jnp.float32),
            pltpu.VMEM((BPW,), jnp.float32),
            pltpu.VMEM((16,), jnp.float32),
            pltpu.VMEM((BPW,), jnp.float32),
            pltpu.SemaphoreType.DMA,
            pltpu.SemaphoreType.DMA,
        ],
    )
    dot = pl.kernel(
        _dot_body,
        out_type=jax.ShapeDtypeStruct((B,), jnp.float32),
        mesh=mesh,
        compiler_params=pltpu.CompilerParams(needs_layout_passes=False,
                                             use_tc_tiling_on_sc=True),
        scratch_types=[
            pltpu.VMEM((BPW,), jnp.int32),
            pltpu.VMEM((BPW,), jnp.int32),
            pltpu.VMEM((2 * G * 8, D), jnp.float32),
            pltpu.VMEM((2 * G * 8, D), jnp.float32),
            pltpu.VMEM((BPW,), jnp.float32),
            pltpu.VMEM((BPW // 4, 128), jnp.float32),
            pltpu.VMEM((BPW,), jnp.float32),
            pltpu.SemaphoreType.DMA,
            pltpu.SemaphoreType.DMA,
            pltpu.SemaphoreType.DMA,
            pltpu.SemaphoreType.DMA,
        ],
    )
    midx = x[:, 1]
    uidx = x[:, 0]
    pb = bias(midx, uidx, mu.reshape(-1), b_movie.reshape(-1),
              b_user.reshape(-1))
    return dot(midx, uidx, W_movie, W_user, pb)


# speedup vs baseline: 1.0265x; 1.0265x over previous
import jax
import jax.numpy as jnp
from jax import lax
from jax.experimental import pallas as pl
from jax.experimental.pallas import tpu as pltpu
from jax.experimental.pallas import tpu_sc as plsc

B = 16384
D = 64
NC = 2
NS = 16
NW = NC * NS
BPW = B // NW
PP = 17
G = 16
NG = BPW // G


def _bias_body(midx_hbm, uidx_hbm, mu_hbm, bm_hbm, bu_hbm, out_hbm,
               midx_v, uidx_v, bm_v, bu_v, mu_v, out_v, sem_bm, sem_bu):
    wid = lax.axis_index("s") * NC + lax.axis_index("c")
    base = wid * BPW
    pltpu.sync_copy(midx_hbm.at[pl.ds(base, BPW)], midx_v)
    pltpu.sync_copy(uidx_hbm.at[pl.ds(base, BPW)], uidx_v)
    pltpu.sync_copy(mu_hbm, mu_v.at[pl.ds(0, 1)])
    cbm = pltpu.async_copy(bm_hbm.at[midx_v], bm_v, sem_bm)
    cbu = pltpu.async_copy(bu_hbm.at[uidx_v], bu_v, sem_bu)
    cbm.wait()
    cbu.wait()
    mu_s = mu_v[...][0]

    def blk(j, _):
        rb = j * 16
        out_v[pl.ds(rb, 16)] = mu_s + (bm_v[pl.ds(rb, 16)] +
                                       bu_v[pl.ds(rb, 16)])
        return 0

    lax.fori_loop(0, BPW // 16, blk, 0, unroll=4)
    pltpu.sync_copy(out_v, out_hbm.at[pl.ds(base, BPW)])


def _dot_body(midx_hbm, uidx_hbm, wm_hbm, wu_hbm, pb_hbm, out_hbm,
              midx_v, uidx_v, tm_v, tu_v, pb_v, part_v, out_v,
              sem_m0, sem_m1, sem_m2, sem_u0, sem_u1, sem_u2):
    wid = lax.axis_index("s") * NC + lax.axis_index("c")
    base = wid * BPW

    pltpu.sync_copy(midx_hbm.at[pl.ds(base, BPW)], midx_v)
    pltpu.sync_copy(uidx_hbm.at[pl.ds(base, BPW)], uidx_v)
    pltpu.sync_copy(pb_hbm.at[pl.ds(base, BPW)], pb_v)

    sems_m = (sem_m0, sem_m1, sem_m2)
    sems_u = (sem_u0, sem_u1, sem_u2)

    def issue(g, slot):
        mv = midx_v[pl.ds(g * G, G)]
        uv = uidx_v[pl.ds(g * G, G)]
        for k in range(G):
            tm0 = lax.shift_right_logical(mv[k], 3) * 8
            tu0 = lax.shift_right_logical(uv[k], 3) * 8
            d0 = (slot * G + k) * 8
            pltpu.async_copy(wm_hbm.at[pl.ds(tm0, 8)],
                             tm_v.at[pl.ds(d0, 8)], sems_m[slot])
            pltpu.async_copy(wu_hbm.at[pl.ds(tu0, 8)],
                             tu_v.at[pl.ds(d0, 8)], sems_u[slot])

    def drain(slot):
        d0 = slot * G * 8
        pltpu.make_async_copy(wm_hbm.at[pl.ds(0, G * 8)],
                              tm_v.at[pl.ds(d0, G * 8)], sems_m[slot]).wait()
        pltpu.make_async_copy(wu_hbm.at[pl.ds(0, G * 8)],
                              tu_v.at[pl.ds(d0, G * 8)], sems_u[slot]).wait()

    def compute(g, slot):
        mv = midx_v[pl.ds(g * G, G)]
        uv = uidx_v[pl.ds(g * G, G)]
        for k in range(G):
            sm = (mv[k] & 7) + (slot * G + k) * 8
            su = (uv[k] & 7) + (slot * G + k) * 8
            a0 = tm_v[sm, pl.ds(0, 16)] * tu_v[su, pl.ds(0, 16)]
            a1 = tm_v[sm, pl.ds(16, 16)] * tu_v[su, pl.ds(16, 16)]
            a2 = tm_v[sm, pl.ds(32, 16)] * tu_v[su, pl.ds(32, 16)]
            a3 = tm_v[sm, pl.ds(48, 16)] * tu_v[su, pl.ds(48, 16)]
            part_v[g * 4 + k // 4, pl.ds((k % 4) * 16, 16)] = \
                (a0 + a1) + (a2 + a3)

    issue(0, 0)
    issue(1, 1)
    issue(2, 2)

    def grp(gg, _):
        g0 = gg * 3
        for j in range(3):
            drain(j)
            compute(g0 + j, j)
            issue(g0 + j + 3, j)
        return 0

    lax.fori_loop(0, (NG - 3) // 3, grp, 0, unroll=1)

    drain(0)
    compute(NG - 5, 0)
    issue(NG - 2, 0)
    drain(1)
    compute(NG - 4, 1)
    issue(NG - 1, 1)
    drain(2)
    compute(NG - 3, 2)
    drain(0)
    compute(NG - 2, 0)
    drain(1)
    compute(NG - 1, 1)

    iota16 = lax.iota(jnp.int32, 16)
    zero16 = jnp.zeros((16,), jnp.int32)

    def blk(j, _):
        rb = j * 16
        rv = rb + iota16
        prow = lax.shift_right_logical(rv, 2)
        pcol = (rv & 3) * 16
        acc0 = pb_v[pl.ds(rb, 16)]
        acc2 = plsc.load_gather(part_v, [prow, pcol])
        acc3 = plsc.load_gather(part_v, [prow, pcol + 1])
        for c in range(2, 16, 2):
            acc2 = acc2 + plsc.load_gather(part_v, [prow, pcol + c])
            acc3 = acc3 + plsc.load_gather(part_v, [prow, pcol + (c + 1)])
        out_v[pl.ds(rb, 16)] = acc0 + (acc2 + acc3)
        return 0

    lax.fori_loop(0, BPW // 16, blk, 0, unroll=2)

    pltpu.sync_copy(out_v, out_hbm.at[pl.ds(base, BPW)])


@jax.jit
def kernel(x, W_movie, W_user, mu, b_movie, b_user):
    mesh = plsc.VectorSubcoreMesh(core_axis_name="c", subcore_axis_name="s",
                                  num_cores=NC, num_subcores=NS)
    bias = pl.kernel(
        _bias_body,
        out_type=jax.ShapeDtypeStruct((B,), jnp.float32),
        mesh=mesh,
        compiler_params=pltpu.CompilerParams(needs_layout_passes=False,
                                             use_tc_tiling_on_sc=False),
        scratch_types=[
            pltpu.VMEM((BPW,), jnp.int32),
            pltpu.VMEM((BPW,), jnp.int32),
            pltpu.VMEM((BPW,), jnp.float32),
            pltpu.VMEM((BPW,), jnp.float32),
            pltpu.VMEM((16,), jnp.float32),
            pltpu.VMEM((BPW,), jnp.float32),
            pltpu.SemaphoreType.DMA,
            pltpu.SemaphoreType.DMA,
        ],
    )
    dot = pl.kernel(
        _dot_body,
        out_type=jax.ShapeDtypeStruct((B,), jnp.float32),
        mesh=mesh,
        compiler_params=pltpu.CompilerParams(needs_layout_passes=False,
                                             use_tc_tiling_on_sc=True),
        scratch_types=[
            pltpu.VMEM((BPW,), jnp.int32),
            pltpu.VMEM((BPW,), jnp.int32),
            pltpu.VMEM((3 * G * 8, D), jnp.float32),
            pltpu.VMEM((3 * G * 8, D), jnp.float32),
            pltpu.VMEM((BPW,), jnp.float32),
            pltpu.VMEM((BPW // 4, 128), jnp.float32),
            pltpu.VMEM((BPW,), jnp.float32),
            pltpu.SemaphoreType.DMA,
            pltpu.SemaphoreType.DMA,
            pltpu.SemaphoreType.DMA,
            pltpu.SemaphoreType.DMA,
            pltpu.SemaphoreType.DMA,
            pltpu.SemaphoreType.DMA,
        ],
    )
    midx = x[:, 1]
    uidx = x[:, 0]
    pb = bias(midx, uidx, mu.reshape(-1), b_movie.reshape(-1),
              b_user.reshape(-1))
    return dot(midx, uidx, W_movie, W_user, pb)


# speedup vs baseline: 1.0739x; 1.0462x over previous
import jax
import jax.numpy as jnp
from jax import lax
from jax.experimental import pallas as pl
from jax.experimental.pallas import tpu as pltpu
from jax.experimental.pallas import tpu_sc as plsc

B = 16384
D = 64
NC = 2
NS = 16
NW = NC * NS
BPW = B // NW
PP = 17


def _body(midx_hbm, uidx_hbm, wm_hbm, wu_hbm, mu_hbm, bm_hbm, bu_hbm, out_hbm,
          midx_v, uidx_v, rows_m, rows_u, bm_v, bu_v, mu_v, part_v,
          out_v, sem_m, sem_u, sem_bm, sem_bu):
    wid = lax.axis_index("s") * NC + lax.axis_index("c")
    base = wid * BPW

    pltpu.sync_copy(midx_hbm.at[pl.ds(base, BPW)], midx_v)
    pltpu.sync_copy(uidx_hbm.at[pl.ds(base, BPW)], uidx_v)
    pltpu.sync_copy(mu_hbm, mu_v.at[pl.ds(0, 1)])

    iota16 = lax.iota(jnp.int32, 16)
    zero16 = jnp.zeros((16,), jnp.int32)

    cm = pltpu.async_copy(wm_hbm.at[midx_v], rows_m, sem_m)
    cu = pltpu.async_copy(wu_hbm.at[uidx_v], rows_u, sem_u)
    cbm = pltpu.async_copy(bm_hbm.at[midx_v], bm_v, sem_bm)
    cbu = pltpu.async_copy(bu_hbm.at[uidx_v], bu_v, sem_bu)
    cm.wait()
    cu.wait()
    cbm.wait()
    cbu.wait()

    def row(r, _):
        a0 = rows_m[r, pl.ds(0, 16)] * rows_u[r, pl.ds(0, 16)]
        a1 = rows_m[r, pl.ds(16, 16)] * rows_u[r, pl.ds(16, 16)]
        a2 = rows_m[r, pl.ds(32, 16)] * rows_u[r, pl.ds(32, 16)]
        a3 = rows_m[r, pl.ds(48, 16)] * rows_u[r, pl.ds(48, 16)]
        part_v[r, pl.ds(0, 16)] = (a0 + a1) + (a2 + a3)
        return 0

    lax.fori_loop(0, BPW, row, 0, unroll=8)

    mu_s = mu_v[...][0]

    def blk(j, _):
        rb = j * 16
        rows = rb + iota16
        acc0 = bm_v[pl.ds(rb, 16)]
        acc1 = bu_v[pl.ds(rb, 16)]
        acc2 = mu_s + plsc.load_gather(part_v, [rows, zero16])
        acc3 = plsc.load_gather(part_v, [rows, zero16 + 1])
        for c in range(2, 16, 2):
            acc2 = acc2 + plsc.load_gather(part_v, [rows, zero16 + c])
            acc3 = acc3 + plsc.load_gather(part_v, [rows, zero16 + (c + 1)])
        out_v[pl.ds(rb, 16)] = (acc0 + acc1) + (acc2 + acc3)
        return 0

    lax.fori_loop(0, BPW // 16, blk, 0, unroll=2)

    pltpu.sync_copy(out_v, out_hbm.at[pl.ds(base, BPW)])


@jax.jit
def kernel(x, W_movie, W_user, mu, b_movie, b_user):
    mesh = plsc.VectorSubcoreMesh(core_axis_name="c", subcore_axis_name="s",
                                  num_cores=NC, num_subcores=NS)
    run = pl.kernel(
        _body,
        out_type=jax.ShapeDtypeStruct((B,), jnp.float32),
        mesh=mesh,
        compiler_params=pltpu.CompilerParams(needs_layout_passes=False,
                                             use_tc_tiling_on_sc=False),
        scratch_types=[
            pltpu.VMEM((BPW,), jnp.int32),
            pltpu.VMEM((BPW,), jnp.int32),
            pltpu.VMEM((BPW, D), jnp.float32),
            pltpu.VMEM((BPW, D), jnp.float32),
            pltpu.VMEM((BPW,), jnp.float32),
            pltpu.VMEM((BPW,), jnp.float32),
            pltpu.VMEM((16,), jnp.float32),
            pltpu.VMEM((BPW, PP), jnp.float32),
            pltpu.VMEM((BPW,), jnp.float32),
            pltpu.SemaphoreType.DMA,
            pltpu.SemaphoreType.DMA,
            pltpu.SemaphoreType.DMA,
            pltpu.SemaphoreType.DMA,
        ],
    )
    return run(x[:, 1], x[:, 0], W_movie, W_user, mu.reshape(-1),
               b_movie.reshape(-1), b_user.reshape(-1))
